# Initial kernel scaffold; baseline (speedup 1.0000x reference)
import jax
import jax.numpy as jnp
from jax.experimental import pallas as pl


def kernel(scores, labels, group_sizes):
    raise NotImplementedError("write your pallas kernel here")


# speedup vs baseline: 2.0012x; 2.0012x over previous
import jax
import jax.numpy as jnp
from jax.experimental import pallas as pl
from jax.experimental.pallas import tpu as pltpu


def _rank_loss_kernel(scol_ref, lcol_ref, srow_ref, lrow_ref, out_ref, acc_ref):
    b = pl.program_id(0)
    t = pl.program_id(1)
    nb = pl.num_programs(0)
    nt = pl.num_programs(1)

    @pl.when(jnp.logical_and(b == 0, t == 0))
    def _init_totals():
        acc_ref[2] = 0.0
        acc_ref[3] = 0.0

    @pl.when(t == 0)
    def _init_group():
        acc_ref[0] = 0.0
        acc_ref[1] = 0.0

    scol = scol_ref[...]
    lcol = lcol_ref[...]
    srow = srow_ref[0]
    lrow = lrow_ref[0]

    x = srow - scol
    sp = jnp.maximum(x, 0.0) + jnp.log1p(jnp.exp(-jnp.abs(x)))
    mask = lcol > lrow
    acc_ref[0] += jnp.sum(jnp.where(mask, sp, 0.0))
    acc_ref[1] += jnp.sum(jnp.where(mask, 1.0, 0.0))

    @pl.when(t == nt - 1)
    def _finalize_group():
        n_pairs = acc_ref[1]
        safe_n = jnp.where(n_pairs > 0, n_pairs, 1.0)
        acc_ref[2] += jnp.where(n_pairs > 0, acc_ref[0] / safe_n, 0.0)
        acc_ref[3] += jnp.where(n_pairs > 0, 1.0, 0.0)

        @pl.when(b == nb - 1)
        def _finalize_output():
            count = acc_ref[3]
            safe_c = jnp.where(count > 0, count, 1.0)
            out_ref[0, 0] = jnp.where(count > 0, acc_ref[2] / safe_c, 0.0)


def kernel(scores, labels, group_sizes):
    scores = scores.reshape(-1)
    labels = labels.reshape(-1)
    n = scores.shape[0]
    num_groups = group_sizes.shape[0]
    g = n // num_groups
    tr = 256
    nt = g // tr

    scores_2d = scores.reshape(num_groups, 1, g)
    labels_2d = labels.reshape(num_groups, 1, g)
    scores_col = scores.reshape(n, 1)
    labels_col = labels.reshape(n, 1)

    out = pl.pallas_call(
        _rank_loss_kernel,
        grid=(num_groups, nt),
        in_specs=[
            pl.BlockSpec((tr, 1), lambda b, t: (b * nt + t, 0)),
            pl.BlockSpec((tr, 1), lambda b, t: (b * nt + t, 0)),
            pl.BlockSpec((1, 1, g), lambda b, t: (b, 0, 0)),
            pl.BlockSpec((1, 1, g), lambda b, t: (b, 0, 0)),
        ],
        out_specs=pl.BlockSpec(memory_space=pltpu.SMEM),
        out_shape=jax.ShapeDtypeStruct((1, 1), jnp.float32),
        scratch_shapes=[pltpu.SMEM((4,), jnp.float32)],
    )(scores_col, labels_col, scores_2d, labels_2d)
    return out[0, 0]
